# baseline (speedup 1.0000x reference)
import jax
import jax.numpy as jnp
from jax import lax
from jax.experimental import pallas as pl
from jax.experimental.pallas import tpu as pltpu
from jax.experimental.pallas import tpu_sc as plsc

_B = 16384
_D = 64
_V = 1000000
_NR = 1000
_NC = 2
_NS = 16
_NW = _NC * _NS
_ROWS_PER_W = _B // _NW
_CHUNK = 128
_NCHUNK = _ROWS_PER_W // _CHUNK
_GROUPS = _CHUNK // 16
_MARGIN = 1.0

_NB = 2048
_SPLIT = 246 * _NB
_RSPLIT = 512


def _make_relayout(d, nb, grid, rows):
    def transpose_exact(x, eye):
        hi = x.astype(jnp.bfloat16).astype(jnp.float32)
        lo = x - hi
        dn = (((0,), (0,)), ((), ()))
        return (lax.dot_general(hi, eye, dn)
                + lax.dot_general(lo, eye, dn))

    def body(x1_ref, x2_ref, o_ref):
        eye = (lax.broadcasted_iota(jnp.int32, (d, d), 0)
               == lax.broadcasted_iota(jnp.int32, (d, d), 1)
               ).astype(jnp.float32)
        xt1 = transpose_exact(x1_ref[...], eye)
        xt2 = transpose_exact(x2_ref[...], eye)
        o_ref[...] = jnp.concatenate([xt1, xt2], axis=1)

    def run(tab_t):
        last_blk = (tab_t.shape[1] - 1) // nb
        return pl.pallas_call(
            body,
            grid=(grid,),
            in_specs=[pl.BlockSpec((d, nb), lambda i: (0, i)),
                      pl.BlockSpec((d, nb),
                                   lambda i: (0, jnp.minimum(grid + i,
                                                             last_blk)))],
            out_specs=pl.BlockSpec((nb, 2 * d), lambda i: (i, 0)),
            out_shape=jax.ShapeDtypeStruct((rows, 2 * d), jnp.float32),
        )(tab_t, tab_t)

    return run


_relayout_ent = _make_relayout(_D, _NB, _SPLIT // _NB, _SPLIT)
_relayout_rel = _make_relayout(_D, _RSPLIT, 1, _RSPLIT)


def _rsqrt(x):
    x = jnp.maximum(x, 1e-30)
    i = plsc.bitcast(x, jnp.int32)
    i = 0x5F3759DF - lax.shift_right_logical(i, 1)
    y = plsc.bitcast(i, jnp.float32)
    for _ in range(3):
        y = y * (1.5 - 0.5 * x * y * y)
    return y


def _sqrt(x):
    x = jnp.maximum(x, 0.0)
    return x * _rsqrt(x)


def _sc_body(phr_h, ptr_h, nhr_h, ntr_h, rlr_h,
             ph_h, pt_h, nh_h, nt_h, rl_h, ent_h, rel_h, out_h,
             phr_i, ptr_i, nhr_i, ntr_i, rlr_i,
             ph_i, pt_i, nh_i, nt_i, rl_i,
             hp_r, tp_r, hn_r, tn_r, rr_r, part_v, sem):
    wid = lax.axis_index("s") * _NC + lax.axis_index("c")
    iota = lax.iota(jnp.int32, 16)
    base = wid * _ROWS_PER_W

    row_scr = [phr_i, ptr_i, nhr_i, ntr_i, rlr_i]
    row_hbm = [phr_h, ptr_h, nhr_h, ntr_h, rlr_h]
    orig_scr = [ph_i, pt_i, nh_i, nt_i, rl_i]
    orig_hbm = [ph_h, pt_h, nh_h, nt_h, rl_h]
    cps = []
    for scr, hbm in zip(row_scr, row_hbm):
        for c in range(_NCHUNK):
            cps.append(pltpu.async_copy(
                hbm.at[pl.ds(base + c * _CHUNK, _CHUNK)], scr.at[c], sem))
    for scr, hbm in zip(orig_scr, orig_hbm):
        cps.append(pltpu.async_copy(
            hbm.at[pl.ds(base, _ROWS_PER_W)], scr, sem))
    for cp in cps:
        cp.wait()

    splits = [_SPLIT, _SPLIT, _SPLIT, _SPLIT, _RSPLIT]

    acc = jnp.zeros((16,), jnp.float32)
    for c in range(_NCHUNK):
        row_cp = [pltpu.async_copy(ent_h.at[phr_i.at[c]], hp_r, sem),
                  pltpu.async_copy(ent_h.at[ptr_i.at[c]], tp_r, sem),
                  pltpu.async_copy(ent_h.at[nhr_i.at[c]], hn_r, sem),
                  pltpu.async_copy(ent_h.at[ntr_i.at[c]], tn_r, sem),
                  pltpu.async_copy(rel_h.at[rlr_i.at[c]], rr_r, sem)]
        for cp in row_cp:
            cp.wait()

        def group_body(g, acc_in, c=c):
            lane = g * 16 + iota
            gidx = c * _CHUNK + g * 16 + iota
            halves = []
            for scr, spl in zip(orig_scr, splits):
                iv = plsc.load_gather(scr, [gidx])
                halves.append(jnp.where(iv >= spl, 64, 0).astype(jnp.int32))
            z = jnp.zeros((16,), jnp.float32)

            def j_body(j, ss):
                rot = lax.bitwise_and(iota + j, _D - 1)
                hp = plsc.load_gather(hp_r, [lane, halves[0] + rot])
                tp = plsc.load_gather(tp_r, [lane, halves[1] + rot])
                hn = plsc.load_gather(hn_r, [lane, halves[2] + rot])
                tn = plsc.load_gather(tn_r, [lane, halves[3] + rot])
                rr = plsc.load_gather(rr_r, [lane, halves[4] + rot])
                return (ss[0] + hp * hp, ss[1] + tp * tp, ss[2] + rr * rr,
                        ss[3] + hp * rr, ss[4] + hp * tp, ss[5] + rr * tp,
                        ss[6] + hn * hn, ss[7] + tn * tn, ss[8] + hn * rr,
                        ss[9] + hn * tn, ss[10] + rr * tn)

            (s_hphp, s_tptp, s_rr, s_hpr, s_hptp, s_rtp,
             s_hnhn, s_tntn, s_hnr, s_hntn, s_rtn) = lax.fori_loop(
                 0, _D, j_body, (z,) * 11)
            ar = _rsqrt(s_rr)
            ahp = _rsqrt(s_hphp)
            atp = _rsqrt(s_tptp)
            ahn = _rsqrt(s_hnhn)
            atn = _rsqrt(s_tntn)
            d2p = (s_hphp * ahp * ahp + s_rr * ar * ar + s_tptp * atp * atp
                   + 2.0 * (s_hpr * ahp * ar - s_hptp * ahp * atp
                            - s_rtp * ar * atp))
            d2n = (s_hnhn * ahn * ahn + s_rr * ar * ar + s_tntn * atn * atn
                   + 2.0 * (s_hnr * ahn * ar - s_hntn * ahn * atn
                            - s_rtn * ar * atn))
            e_pos = _sqrt(d2p)
            e_neg = _sqrt(d2n)
            return acc_in + jnp.maximum(_MARGIN + e_pos - e_neg, 0.0)

        acc = lax.fori_loop(0, _GROUPS, group_body, acc)
    part_v[...] = acc
    pltpu.sync_copy(part_v, out_h.at[wid])


_sc_kernel = pl.kernel(
    _sc_body,
    out_type=jax.ShapeDtypeStruct((_NW, 16), jnp.float32),
    mesh=plsc.VectorSubcoreMesh(core_axis_name="c", subcore_axis_name="s",
                                num_cores=_NC, num_subcores=_NS),
    scratch_types=[pltpu.VMEM((_NCHUNK, _CHUNK), jnp.int32) for _ in range(5)]
    + [pltpu.VMEM((_ROWS_PER_W,), jnp.int32) for _ in range(5)]
    + [pltpu.VMEM((_CHUNK, 2 * _D), jnp.float32) for _ in range(5)]
    + [pltpu.VMEM((16,), jnp.float32), pltpu.SemaphoreType.DMA],
    compiler_params=pltpu.CompilerParams(needs_layout_passes=False,
                                         use_tc_tiling_on_sc=True),
)


def _final_body(p_ref, o_ref):
    o_ref[...] = jnp.reshape(jnp.sum(p_ref[...]) * (1.0 / _B), (1, 1))


def kernel(pos_pairs, neg_pairs, rels, ent_emb, rel_emb):
    ph = pos_pairs[:, 0].astype(jnp.int32)
    pt = pos_pairs[:, 1].astype(jnp.int32)
    nh = neg_pairs[:, 0].astype(jnp.int32)
    nt = neg_pairs[:, 1].astype(jnp.int32)
    rl = rels[:, 0].astype(jnp.int32)
    idx = [ph, pt, nh, nt]
    rows = [jnp.where(x < _SPLIT, x, x - _SPLIT) for x in idx]
    rows.append(jnp.where(rl < _RSPLIT, rl, rl - _RSPLIT))
    ent_tab = _relayout_ent(ent_emb.T)
    rel_tab = _relayout_rel(rel_emb.T)
    partials = _sc_kernel(*rows, ph, pt, nh, nt, rl, ent_tab, rel_tab)
    total = pl.pallas_call(
        _final_body,
        out_shape=jax.ShapeDtypeStruct((1, 1), jnp.float32),
    )(partials)
    return total[0, 0]


# speedup vs baseline: 1.2557x; 1.2557x over previous
import jax
import jax.numpy as jnp
from jax import lax
from jax.experimental import pallas as pl
from jax.experimental.pallas import tpu as pltpu
from jax.experimental.pallas import tpu_sc as plsc

_B = 16384
_D = 64
_V = 1000000
_NR = 1000
_NC = 2
_NS = 16
_NW = _NC * _NS
_ROWS_PER_W = _B // _NW
_CHUNK = 128
_NCHUNK = _ROWS_PER_W // _CHUNK
_GROUPS = _CHUNK // 16
_MARGIN = 1.0

_NB = 8192
_SPLIT = 62 * _NB
_RSPLIT = 512


def _make_relayout(d, nb, grid, rows):
    def transpose_exact(x, eye):
        hi = x.astype(jnp.bfloat16).astype(jnp.float32)
        lo = x - hi
        dn = (((0,), (0,)), ((), ()))
        return (lax.dot_general(hi, eye, dn)
                + lax.dot_general(lo, eye, dn))

    def body(x1_ref, x2_ref, o_ref):
        eye = (lax.broadcasted_iota(jnp.int32, (d, d), 0)
               == lax.broadcasted_iota(jnp.int32, (d, d), 1)
               ).astype(jnp.float32)
        xt1 = transpose_exact(x1_ref[...], eye)
        xt2 = transpose_exact(x2_ref[...], eye)
        o_ref[...] = jnp.concatenate([xt1, xt2], axis=1)

    def run(tab_t):
        last_blk = (tab_t.shape[1] - 1) // nb
        return pl.pallas_call(
            body,
            grid=(grid,),
            in_specs=[pl.BlockSpec((d, nb), lambda i: (0, i)),
                      pl.BlockSpec((d, nb),
                                   lambda i: (0, jnp.minimum(grid + i,
                                                             last_blk)))],
            out_specs=pl.BlockSpec((nb, 2 * d), lambda i: (i, 0)),
            out_shape=jax.ShapeDtypeStruct((rows, 2 * d), jnp.float32),
        )(tab_t, tab_t)

    return run


_relayout_ent = _make_relayout(_D, _NB, _SPLIT // _NB, _SPLIT)
_relayout_rel = _make_relayout(_D, _RSPLIT, 1, _RSPLIT)


def _rsqrt(x):
    x = jnp.maximum(x, 1e-30)
    i = plsc.bitcast(x, jnp.int32)
    i = 0x5F3759DF - lax.shift_right_logical(i, 1)
    y = plsc.bitcast(i, jnp.float32)
    for _ in range(3):
        y = y * (1.5 - 0.5 * x * y * y)
    return y


def _sqrt(x):
    x = jnp.maximum(x, 0.0)
    return x * _rsqrt(x)


def _sc_body(phr_h, ptr_h, nhr_h, ntr_h, rlr_h,
             ph_h, pt_h, nh_h, nt_h, rl_h, ent_h, rel_h, out_h,
             phr_i, ptr_i, nhr_i, ntr_i, rlr_i,
             ph_i, pt_i, nh_i, nt_i, rl_i,
             hp_r, tp_r, hn_r, tn_r, rr_r, part_v, sem):
    wid = lax.axis_index("s") * _NC + lax.axis_index("c")
    iota = lax.iota(jnp.int32, 16)
    base = wid * _ROWS_PER_W

    row_scr = [phr_i, ptr_i, nhr_i, ntr_i, rlr_i]
    row_hbm = [phr_h, ptr_h, nhr_h, ntr_h, rlr_h]
    orig_scr = [ph_i, pt_i, nh_i, nt_i, rl_i]
    orig_hbm = [ph_h, pt_h, nh_h, nt_h, rl_h]
    cps = []
    for scr, hbm in zip(row_scr, row_hbm):
        for c in range(_NCHUNK):
            cps.append(pltpu.async_copy(
                hbm.at[pl.ds(base + c * _CHUNK, _CHUNK)], scr.at[c], sem))
    for scr, hbm in zip(orig_scr, orig_hbm):
        cps.append(pltpu.async_copy(
            hbm.at[pl.ds(base, _ROWS_PER_W)], scr, sem))
    for cp in cps:
        cp.wait()

    splits = [_SPLIT, _SPLIT, _SPLIT, _SPLIT, _RSPLIT]

    acc = jnp.zeros((16,), jnp.float32)
    for c in range(_NCHUNK):
        row_cp = [pltpu.async_copy(ent_h.at[phr_i.at[c]], hp_r, sem),
                  pltpu.async_copy(ent_h.at[ptr_i.at[c]], tp_r, sem),
                  pltpu.async_copy(ent_h.at[nhr_i.at[c]], hn_r, sem),
                  pltpu.async_copy(ent_h.at[ntr_i.at[c]], tn_r, sem),
                  pltpu.async_copy(rel_h.at[rlr_i.at[c]], rr_r, sem)]
        for cp in row_cp:
            cp.wait()

        def group_body(g, acc_in, c=c):
            lane = g * 16 + iota
            gidx = c * _CHUNK + g * 16 + iota
            halves = []
            for scr, spl in zip(orig_scr, splits):
                iv = plsc.load_gather(scr, [gidx])
                halves.append(jnp.where(iv >= spl, 64, 0).astype(jnp.int32))
            z = jnp.zeros((16,), jnp.float32)

            def j_body(j, ss):
                rot = lax.bitwise_and(iota + j, _D - 1)
                hp = plsc.load_gather(hp_r, [lane, halves[0] + rot])
                tp = plsc.load_gather(tp_r, [lane, halves[1] + rot])
                hn = plsc.load_gather(hn_r, [lane, halves[2] + rot])
                tn = plsc.load_gather(tn_r, [lane, halves[3] + rot])
                rr = plsc.load_gather(rr_r, [lane, halves[4] + rot])
                return (ss[0] + hp * hp, ss[1] + tp * tp, ss[2] + rr * rr,
                        ss[3] + hp * rr, ss[4] + hp * tp, ss[5] + rr * tp,
                        ss[6] + hn * hn, ss[7] + tn * tn, ss[8] + hn * rr,
                        ss[9] + hn * tn, ss[10] + rr * tn)

            (s_hphp, s_tptp, s_rr, s_hpr, s_hptp, s_rtp,
             s_hnhn, s_tntn, s_hnr, s_hntn, s_rtn) = lax.fori_loop(
                 0, _D, j_body, (z,) * 11)
            ar = _rsqrt(s_rr)
            ahp = _rsqrt(s_hphp)
            atp = _rsqrt(s_tptp)
            ahn = _rsqrt(s_hnhn)
            atn = _rsqrt(s_tntn)
            d2p = (s_hphp * ahp * ahp + s_rr * ar * ar + s_tptp * atp * atp
                   + 2.0 * (s_hpr * ahp * ar - s_hptp * ahp * atp
                            - s_rtp * ar * atp))
            d2n = (s_hnhn * ahn * ahn + s_rr * ar * ar + s_tntn * atn * atn
                   + 2.0 * (s_hnr * ahn * ar - s_hntn * ahn * atn
                            - s_rtn * ar * atn))
            e_pos = _sqrt(d2p)
            e_neg = _sqrt(d2n)
            return acc_in + jnp.maximum(_MARGIN + e_pos - e_neg, 0.0)

        acc = lax.fori_loop(0, _GROUPS, group_body, acc)
    part_v[...] = acc
    pltpu.sync_copy(part_v, out_h.at[wid])


_sc_kernel = pl.kernel(
    _sc_body,
    out_type=jax.ShapeDtypeStruct((_NW, 16), jnp.float32),
    mesh=plsc.VectorSubcoreMesh(core_axis_name="c", subcore_axis_name="s",
                                num_cores=_NC, num_subcores=_NS),
    scratch_types=[pltpu.VMEM((_NCHUNK, _CHUNK), jnp.int32) for _ in range(5)]
    + [pltpu.VMEM((_ROWS_PER_W,), jnp.int32) for _ in range(5)]
    + [pltpu.VMEM((_CHUNK, 2 * _D), jnp.float32) for _ in range(5)]
    + [pltpu.VMEM((16,), jnp.float32), pltpu.SemaphoreType.DMA],
    compiler_params=pltpu.CompilerParams(needs_layout_passes=False,
                                         use_tc_tiling_on_sc=True),
)


def _final_body(p_ref, o_ref):
    o_ref[...] = jnp.reshape(jnp.sum(p_ref[...]) * (1.0 / _B), (1, 1))


def kernel(pos_pairs, neg_pairs, rels, ent_emb, rel_emb):
    ph = pos_pairs[:, 0].astype(jnp.int32)
    pt = pos_pairs[:, 1].astype(jnp.int32)
    nh = neg_pairs[:, 0].astype(jnp.int32)
    nt = neg_pairs[:, 1].astype(jnp.int32)
    rl = rels[:, 0].astype(jnp.int32)
    idx = [ph, pt, nh, nt]
    rows = [jnp.where(x < _SPLIT, x, x - _SPLIT) for x in idx]
    rows.append(jnp.where(rl < _RSPLIT, rl, rl - _RSPLIT))
    ent_tab = _relayout_ent(ent_emb.T)
    rel_tab = _relayout_rel(rel_emb.T)
    partials = _sc_kernel(*rows, ph, pt, nh, nt, rl, ent_tab, rel_tab)
    total = pl.pallas_call(
        _final_body,
        out_shape=jax.ShapeDtypeStruct((1, 1), jnp.float32),
    )(partials)
    return total[0, 0]


# speedup vs baseline: 1.7445x; 1.3893x over previous
import jax
import jax.numpy as jnp
from jax import lax
from jax.experimental import pallas as pl
from jax.experimental.pallas import tpu as pltpu
from jax.experimental.pallas import tpu_sc as plsc

_B = 16384
_D = 64
_V = 1000000
_NR = 1000
_NC = 2
_NS = 16
_NW = _NC * _NS
_ROWS_PER_W = _B // _NW
_CHUNK = 128
_NCHUNK = _ROWS_PER_W // _CHUNK
_GROUPS = _CHUNK // 16
_MARGIN = 1.0

_NB = 8192
_GRIDE = 31
_S4 = _GRIDE * _NB
_S4R = 256


def _make_relayout(d, nb, grid, s4, ncols):
    last_blk = (ncols - 1) // nb

    def body(x0_ref, x1_ref, x2_ref, x3_ref, o_ref):
        eye = (lax.broadcasted_iota(jnp.int32, (d, d), 0)
               == lax.broadcasted_iota(jnp.int32, (d, d), 1)
               ).astype(jnp.float32)
        dn = (((0,), (0,)), ((), ()))
        xts = [lax.dot_general(r[...], eye, dn).astype(jnp.bfloat16)
               for r in (x0_ref, x1_ref, x2_ref, x3_ref)]
        u = [lax.bitcast_convert_type(x, jnp.uint16).astype(jnp.int32)
             for x in xts]
        w01 = lax.bitwise_or(u[0], lax.shift_left(u[1], 16))
        w23 = lax.bitwise_or(u[2], lax.shift_left(u[3], 16))
        o_ref[...] = jnp.concatenate([w01, w23], axis=1)

    def mkmap(a):
        return lambda i: (0, jnp.minimum(a * grid + i, last_blk))

    def run(tab_t):
        return pl.pallas_call(
            body,
            grid=(grid,),
            in_specs=[pl.BlockSpec((d, nb), mkmap(a)) for a in range(4)],
            out_specs=pl.BlockSpec((nb, 2 * d), lambda i: (i, 0)),
            out_shape=jax.ShapeDtypeStruct((s4, 2 * d), jnp.int32),
        )(tab_t, tab_t, tab_t, tab_t)

    return run


_relayout_ent = _make_relayout(_D, _NB, _GRIDE, _S4, _V)
_relayout_rel = _make_relayout(_D, _S4R, 1, _S4R, _NR)


def _rsqrt(x):
    x = jnp.maximum(x, 1e-30)
    i = plsc.bitcast(x, jnp.int32)
    i = 0x5F3759DF - lax.shift_right_logical(i, 1)
    y = plsc.bitcast(i, jnp.float32)
    for _ in range(3):
        y = y * (1.5 - 0.5 * x * y * y)
    return y


def _sqrt(x):
    x = jnp.maximum(x, 0.0)
    return x * _rsqrt(x)


def _sc_body(phr_h, ptr_h, nhr_h, ntr_h, rlr_h,
             ph_h, pt_h, nh_h, nt_h, rl_h, ent_h, rel_h, out_h,
             phr_i, ptr_i, nhr_i, ntr_i, rlr_i,
             ph_i, pt_i, nh_i, nt_i, rl_i,
             hp_r, tp_r, hn_r, tn_r, rr_r, part_v, sem):
    wid = lax.axis_index("s") * _NC + lax.axis_index("c")
    iota = lax.iota(jnp.int32, 16)
    base = wid * _ROWS_PER_W

    row_scr = [phr_i, ptr_i, nhr_i, ntr_i, rlr_i]
    row_hbm = [phr_h, ptr_h, nhr_h, ntr_h, rlr_h]
    orig_scr = [ph_i, pt_i, nh_i, nt_i, rl_i]
    orig_hbm = [ph_h, pt_h, nh_h, nt_h, rl_h]
    cps = []
    for scr, hbm in zip(row_scr, row_hbm):
        for c in range(_NCHUNK):
            cps.append(pltpu.async_copy(
                hbm.at[pl.ds(base + c * _CHUNK, _CHUNK)], scr.at[c], sem))
    for scr, hbm in zip(orig_scr, orig_hbm):
        cps.append(pltpu.async_copy(
            hbm.at[pl.ds(base, _ROWS_PER_W)], scr, sem))
    for cp in cps:
        cp.wait()

    s4s = [_S4, _S4, _S4, _S4, _S4R]
    mask_hi = jnp.full((16,), -65536, jnp.int32)

    acc = jnp.zeros((16,), jnp.float32)
    for c in range(_NCHUNK):
        row_cp = [pltpu.async_copy(ent_h.at[phr_i.at[c]], hp_r, sem),
                  pltpu.async_copy(ent_h.at[ptr_i.at[c]], tp_r, sem),
                  pltpu.async_copy(ent_h.at[nhr_i.at[c]], hn_r, sem),
                  pltpu.async_copy(ent_h.at[ntr_i.at[c]], tn_r, sem),
                  pltpu.async_copy(rel_h.at[rlr_i.at[c]], rr_r, sem)]
        for cp in row_cp:
            cp.wait()

        def group_body(g, acc_in, c=c):
            lane = g * 16 + iota
            gidx = c * _CHUNK + g * 16 + iota
            lsels, hisels = [], []
            for scr, s4 in zip(orig_scr, s4s):
                iv = plsc.load_gather(scr, [gidx])
                q2 = iv >= (2 * s4)
                lsels.append(jnp.where(q2, 64, 0).astype(jnp.int32))
                hisels.append(jnp.where(q2, iv - 2 * s4, iv) >= s4)
            z = jnp.zeros((16,), jnp.float32)
            bufs = [hp_r, tp_r, hn_r, tn_r, rr_r]

            def j_body(j, ss):
                rot = lax.bitwise_and(iota + j, _D - 1)
                vals = []
                for buf, lsel, hisel in zip(bufs, lsels, hisels):
                    w = plsc.load_gather(buf, [lane, lsel + rot])
                    bits = jnp.where(hisel,
                                     lax.bitwise_and(w, mask_hi),
                                     lax.shift_left(w, 16))
                    vals.append(plsc.bitcast(bits, jnp.float32))
                hp, tp, hn, tn, rr = vals
                return (ss[0] + hp * hp, ss[1] + tp * tp, ss[2] + rr * rr,
                        ss[3] + hp * rr, ss[4] + hp * tp, ss[5] + rr * tp,
                        ss[6] + hn * hn, ss[7] + tn * tn, ss[8] + hn * rr,
                        ss[9] + hn * tn, ss[10] + rr * tn)

            (s_hphp, s_tptp, s_rr, s_hpr, s_hptp, s_rtp,
             s_hnhn, s_tntn, s_hnr, s_hntn, s_rtn) = lax.fori_loop(
                 0, _D, j_body, (z,) * 11)
            ar = _rsqrt(s_rr)
            ahp = _rsqrt(s_hphp)
            atp = _rsqrt(s_tptp)
            ahn = _rsqrt(s_hnhn)
            atn = _rsqrt(s_tntn)
            d2p = (s_hphp * ahp * ahp + s_rr * ar * ar + s_tptp * atp * atp
                   + 2.0 * (s_hpr * ahp * ar - s_hptp * ahp * atp
                            - s_rtp * ar * atp))
            d2n = (s_hnhn * ahn * ahn + s_rr * ar * ar + s_tntn * atn * atn
                   + 2.0 * (s_hnr * ahn * ar - s_hntn * ahn * atn
                            - s_rtn * ar * atn))
            e_pos = _sqrt(d2p)
            e_neg = _sqrt(d2n)
            return acc_in + jnp.maximum(_MARGIN + e_pos - e_neg, 0.0)

        acc = lax.fori_loop(0, _GROUPS, group_body, acc)

    part_v[...] = acc
    pltpu.sync_copy(part_v, out_h.at[wid])


_sc_kernel = pl.kernel(
    _sc_body,
    out_type=jax.ShapeDtypeStruct((_NW, 16), jnp.float32),
    mesh=plsc.VectorSubcoreMesh(core_axis_name="c", subcore_axis_name="s",
                                num_cores=_NC, num_subcores=_NS),
    scratch_types=[pltpu.VMEM((_NCHUNK, _CHUNK), jnp.int32) for _ in range(5)]
    + [pltpu.VMEM((_ROWS_PER_W,), jnp.int32) for _ in range(5)]
    + [pltpu.VMEM((_CHUNK, 2 * _D), jnp.int32) for _ in range(5)]
    + [pltpu.VMEM((16,), jnp.float32), pltpu.SemaphoreType.DMA],
    compiler_params=pltpu.CompilerParams(needs_layout_passes=False,
                                         use_tc_tiling_on_sc=True),
)


def _final_body(p_ref, o_ref):
    o_ref[...] = jnp.reshape(jnp.sum(p_ref[...]) * (1.0 / _B), (1, 1))


def kernel(pos_pairs, neg_pairs, rels, ent_emb, rel_emb):
    ph = pos_pairs[:, 0].astype(jnp.int32)
    pt = pos_pairs[:, 1].astype(jnp.int32)
    nh = neg_pairs[:, 0].astype(jnp.int32)
    nt = neg_pairs[:, 1].astype(jnp.int32)
    rl = rels[:, 0].astype(jnp.int32)
    idx = [ph, pt, nh, nt]
    rows = [x - (x // _S4) * _S4 for x in idx]
    rows.append(rl - (rl // _S4R) * _S4R)
    ent_tab = _relayout_ent(ent_emb.T)
    rel_tab = _relayout_rel(rel_emb.T)
    partials = _sc_kernel(*rows, ph, pt, nh, nt, rl, ent_tab, rel_tab)
    total = pl.pallas_call(
        _final_body,
        out_shape=jax.ShapeDtypeStruct((1, 1), jnp.float32),
    )(partials)
    return total[0, 0]


# speedup vs baseline: 1.8469x; 1.0587x over previous
import jax
import jax.numpy as jnp
from jax import lax
from jax.experimental import pallas as pl
from jax.experimental.pallas import tpu as pltpu
from jax.experimental.pallas import tpu_sc as plsc

_B = 16384
_D = 64
_V = 1000000
_NR = 1000
_NC = 2
_NS = 16
_NW = _NC * _NS
_ROWS_PER_W = _B // _NW
_CHUNK = 64
_NCHUNK = _ROWS_PER_W // _CHUNK
_GROUPS = _CHUNK // 16
_MARGIN = 1.0

_NB = 8192
_GRIDE = 31
_S4 = _GRIDE * _NB
_S4R = 256


def _make_relayout(d, nb, grid, s4, ncols):
    last_blk = (ncols - 1) // nb

    def body(x0_ref, x1_ref, x2_ref, x3_ref, o_ref):
        eye = (lax.broadcasted_iota(jnp.int32, (d, d), 0)
               == lax.broadcasted_iota(jnp.int32, (d, d), 1)
               ).astype(jnp.float32)
        dn = (((0,), (0,)), ((), ()))
        xts = [lax.dot_general(r[...], eye, dn).astype(jnp.bfloat16)
               for r in (x0_ref, x1_ref, x2_ref, x3_ref)]
        u = [lax.bitcast_convert_type(x, jnp.uint16).astype(jnp.int32)
             for x in xts]
        w01 = lax.bitwise_or(u[0], lax.shift_left(u[1], 16))
        w23 = lax.bitwise_or(u[2], lax.shift_left(u[3], 16))
        o_ref[...] = jnp.concatenate([w01, w23], axis=1)

    def mkmap(a):
        return lambda i: (0, jnp.minimum(a * grid + i, last_blk))

    def run(tab_t):
        return pl.pallas_call(
            body,
            grid=(grid,),
            in_specs=[pl.BlockSpec((d, nb), mkmap(a)) for a in range(4)],
            out_specs=pl.BlockSpec((nb, 2 * d), lambda i: (i, 0)),
            out_shape=jax.ShapeDtypeStruct((s4, 2 * d), jnp.int32),
        )(tab_t, tab_t, tab_t, tab_t)

    return run


_relayout_ent = _make_relayout(_D, _NB, _GRIDE, _S4, _V)
_relayout_rel = _make_relayout(_D, _S4R, 1, _S4R, _NR)


def _rsqrt(x):
    x = jnp.maximum(x, 1e-30)
    i = plsc.bitcast(x, jnp.int32)
    i = 0x5F3759DF - lax.shift_right_logical(i, 1)
    y = plsc.bitcast(i, jnp.float32)
    for _ in range(3):
        y = y * (1.5 - 0.5 * x * y * y)
    return y


def _sqrt(x):
    x = jnp.maximum(x, 0.0)
    return x * _rsqrt(x)


def _sc_body(phr_h, ptr_h, nhr_h, ntr_h, rlr_h,
             ph_h, pt_h, nh_h, nt_h, rl_h, ent_h, rel_h, out_h,
             phr_i, ptr_i, nhr_i, ntr_i, rlr_i,
             ph_i, pt_i, nh_i, nt_i, rl_i,
             hp_a, tp_a, hn_a, tn_a, rr_a,
             hp_b, tp_b, hn_b, tn_b, rr_b, part_v, sem):
    wid = lax.axis_index("s") * _NC + lax.axis_index("c")
    iota = lax.iota(jnp.int32, 16)
    base = wid * _ROWS_PER_W

    row_scr = [phr_i, ptr_i, nhr_i, ntr_i, rlr_i]
    row_hbm = [phr_h, ptr_h, nhr_h, ntr_h, rlr_h]
    orig_scr = [ph_i, pt_i, nh_i, nt_i, rl_i]
    orig_hbm = [ph_h, pt_h, nh_h, nt_h, rl_h]
    cps = []
    for scr, hbm in zip(row_scr, row_hbm):
        for c in range(_NCHUNK):
            cps.append(pltpu.async_copy(
                hbm.at[pl.ds(base + c * _CHUNK, _CHUNK)], scr.at[c], sem))
    for scr, hbm in zip(orig_scr, orig_hbm):
        cps.append(pltpu.async_copy(
            hbm.at[pl.ds(base, _ROWS_PER_W)], scr, sem))
    for cp in cps:
        cp.wait()

    s4s = [_S4, _S4, _S4, _S4, _S4R]
    mask_hi = jnp.full((16,), -65536, jnp.int32)
    bufs_ab = [[hp_a, tp_a, hn_a, tn_a, rr_a],
               [hp_b, tp_b, hn_b, tn_b, rr_b]]

    def fire(c, bufs):
        return [pltpu.async_copy(ent_h.at[phr_i.at[c]], bufs[0], sem),
                pltpu.async_copy(ent_h.at[ptr_i.at[c]], bufs[1], sem),
                pltpu.async_copy(ent_h.at[nhr_i.at[c]], bufs[2], sem),
                pltpu.async_copy(ent_h.at[ntr_i.at[c]], bufs[3], sem),
                pltpu.async_copy(rel_h.at[rlr_i.at[c]], bufs[4], sem)]

    acc = jnp.zeros((16,), jnp.float32)
    descs = fire(0, bufs_ab[0])
    for c in range(_NCHUNK):
        cur = bufs_ab[c % 2]
        for cp in descs:
            cp.wait()
        if c + 1 < _NCHUNK:
            descs = fire(c + 1, bufs_ab[(c + 1) % 2])

        def group_body(g, acc_in, c=c, bufs=cur):
            lane = g * 16 + iota
            gidx = c * _CHUNK + g * 16 + iota
            lsels, hisels = [], []
            for scr, s4 in zip(orig_scr, s4s):
                iv = plsc.load_gather(scr, [gidx])
                q2 = iv >= (2 * s4)
                lsels.append(jnp.where(q2, 64, 0).astype(jnp.int32))
                hisels.append(jnp.where(q2, iv - 2 * s4, iv) >= s4)
            z = jnp.zeros((16,), jnp.float32)

            def j_body(j, ss):
                for dj in range(2):
                    rot = lax.bitwise_and(iota + (2 * j + dj), _D - 1)
                    vals = []
                    for buf, lsel, hisel in zip(bufs, lsels, hisels):
                        w = plsc.load_gather(buf, [lane, lsel + rot])
                        bits = jnp.where(hisel,
                                         lax.bitwise_and(w, mask_hi),
                                         lax.shift_left(w, 16))
                        vals.append(plsc.bitcast(bits, jnp.float32))
                    hp, tp, hn, tn, rr = vals
                    ss = (ss[0] + hp * hp, ss[1] + tp * tp, ss[2] + rr * rr,
                          ss[3] + hp * rr, ss[4] + hp * tp, ss[5] + rr * tp,
                          ss[6] + hn * hn, ss[7] + tn * tn, ss[8] + hn * rr,
                          ss[9] + hn * tn, ss[10] + rr * tn)
                return ss

            (s_hphp, s_tptp, s_rr, s_hpr, s_hptp, s_rtp,
             s_hnhn, s_tntn, s_hnr, s_hntn, s_rtn) = lax.fori_loop(
                 0, _D // 2, j_body, (z,) * 11)
            ar = _rsqrt(s_rr)
            ahp = _rsqrt(s_hphp)
            atp = _rsqrt(s_tptp)
            ahn = _rsqrt(s_hnhn)
            atn = _rsqrt(s_tntn)
            d2p = (s_hphp * ahp * ahp + s_rr * ar * ar + s_tptp * atp * atp
                   + 2.0 * (s_hpr * ahp * ar - s_hptp * ahp * atp
                            - s_rtp * ar * atp))
            d2n = (s_hnhn * ahn * ahn + s_rr * ar * ar + s_tntn * atn * atn
                   + 2.0 * (s_hnr * ahn * ar - s_hntn * ahn * atn
                            - s_rtn * ar * atn))
            e_pos = _sqrt(d2p)
            e_neg = _sqrt(d2n)
            return acc_in + jnp.maximum(_MARGIN + e_pos - e_neg, 0.0)

        acc = lax.fori_loop(0, _GROUPS, group_body, acc)

    part_v[...] = acc
    pltpu.sync_copy(part_v, out_h.at[wid])


_sc_kernel = pl.kernel(
    _sc_body,
    out_type=jax.ShapeDtypeStruct((_NW, 16), jnp.float32),
    mesh=plsc.VectorSubcoreMesh(core_axis_name="c", subcore_axis_name="s",
                                num_cores=_NC, num_subcores=_NS),
    scratch_types=[pltpu.VMEM((_NCHUNK, _CHUNK), jnp.int32) for _ in range(5)]
    + [pltpu.VMEM((_ROWS_PER_W,), jnp.int32) for _ in range(5)]
    + [pltpu.VMEM((_CHUNK, 2 * _D), jnp.int32) for _ in range(10)]
    + [pltpu.VMEM((16,), jnp.float32), pltpu.SemaphoreType.DMA],
    compiler_params=pltpu.CompilerParams(needs_layout_passes=False,
                                         use_tc_tiling_on_sc=True),
)


def _final_body(p_ref, o_ref):
    o_ref[...] = jnp.reshape(jnp.sum(p_ref[...]) * (1.0 / _B), (1, 1))


def kernel(pos_pairs, neg_pairs, rels, ent_emb, rel_emb):
    ph = pos_pairs[:, 0].astype(jnp.int32)
    pt = pos_pairs[:, 1].astype(jnp.int32)
    nh = neg_pairs[:, 0].astype(jnp.int32)
    nt = neg_pairs[:, 1].astype(jnp.int32)
    rl = rels[:, 0].astype(jnp.int32)
    idx = [ph, pt, nh, nt]
    rows = [x - (x // _S4) * _S4 for x in idx]
    rows.append(rl - (rl // _S4R) * _S4R)
    ent_tab = _relayout_ent(ent_emb.T)
    rel_tab = _relayout_rel(rel_emb.T)
    partials = _sc_kernel(*rows, ph, pt, nh, nt, rl, ent_tab, rel_tab)
    total = pl.pallas_call(
        _final_body,
        out_shape=jax.ShapeDtypeStruct((1, 1), jnp.float32),
    )(partials)
    return total[0, 0]
